# baseline (speedup 1.0000x reference)
import functools

import jax
import jax.numpy as jnp
from jax import lax
from jax.experimental import pallas as pl
from jax.experimental.pallas import tpu as pltpu
from jax.experimental.pallas import tpu_sc as plsc

NUM_CLASSES = 1024
EMB = 512
N_SUPPORT = 32768
N_QUERY = 16384

NC = 2
NS = 16
L = 16
NW = NC * NS
CPT = NUM_CLASSES // NW
CHUNKS = EMB // L
LCH = 4096
NCH = N_SUPPORT // LCH
GB = 128


def _sc_segment_body(feat_hbm, lbl_hbm, sums_hbm, counts_hbm,
                     lblchunk_v, ilist_v, llist_v, batch_v, acc_v, cnt_v, sem):
    c = lax.axis_index("c")
    s = lax.axis_index("s")
    tid = c * NS + s
    lo = tid * CPT

    zeros = jnp.zeros((L,), jnp.float32)

    def zero_acc(r, _):
        for k in range(CHUNKS):
            acc_v[r, pl.ds(k * L, L)] = zeros
        cnt_v[r, pl.ds(0, L)] = zeros
        return 0
    lax.fori_loop(0, CPT, zero_acc, 0)

    iota16 = lax.iota(jnp.int32, L)

    def chunk_body(ch, _):
        pltpu.sync_copy(lbl_hbm.at[pl.ds(ch * LCH, LCH)], lblchunk_v)

        def prefill(i, _):
            ilist_v[pl.ds(i * L, L)] = jnp.zeros((L,), jnp.int32)
            llist_v[pl.ds(i * L, L)] = jnp.full((L,), -1, jnp.int32)
            return 0
        lax.fori_loop(0, LCH // L, prefill, 0)

        def scan_body(i, n):
            lv = lblchunk_v[pl.ds(i * L, L)]
            m = (lv >= lo) & (lv < lo + CPT)
            plsc.store_compressed(ilist_v.at[pl.ds(n, L)],
                                  iota16 + (ch * LCH + i * L), mask=m)
            plsc.store_compressed(llist_v.at[pl.ds(n, L)], lv, mask=m)
            return n + jnp.sum(jnp.where(m, 1, 0))
        n_match = lax.fori_loop(0, LCH // L, scan_body, 0)

        nbatch = (n_match + (GB - 1)) // GB

        def gather_body(b, _):
            pltpu.async_copy(feat_hbm.at[ilist_v.at[pl.ds(b * GB, GB)]],
                             batch_v, sem).wait()

            def group_body(g, _):
                lvec = llist_v[pl.ds(b * GB + g * L, L)]
                for j in range(L):
                    lab = lvec[j]

                    @pl.when(lab >= lo)
                    def _(lab=lab, j=j):
                        cl = lab - lo
                        r = g * L + j
                        parts = []
                        for k in range(CHUNKS):
                            v = batch_v[r, pl.ds(k * L, L)]
                            parts.append(v * v)
                        while len(parts) > 1:
                            rest = [parts[-1]] if len(parts) % 2 else []
                            parts = [parts[i2] + parts[i2 + 1]
                                     for i2 in range(0, len(parts) - 1, 2)] \
                                + rest
                        tv = jnp.broadcast_to(jnp.sum(parts[0]), (L,))
                        iv = plsc.bitcast(tv, jnp.int32)
                        y = plsc.bitcast(jnp.int32(0x5F3759DF) - (iv >> 1),
                                         jnp.float32)
                        for _ in range(3):
                            y = y * (jnp.float32(1.5)
                                     - jnp.float32(0.5) * tv * y * y)
                        for k in range(CHUNKS):
                            acc_v[cl, pl.ds(k * L, L)] = (
                                acc_v[cl, pl.ds(k * L, L)]
                                + batch_v[r, pl.ds(k * L, L)] * y)
                        cnt_v[cl, pl.ds(0, L)] = (
                            cnt_v[cl, pl.ds(0, L)]
                            + jnp.full((L,), 1.0, jnp.float32))
                return 0

            lax.fori_loop(0, GB // L, group_body, 0)
            return 0

        lax.fori_loop(0, nbatch, gather_body, 0)
        return 0

    lax.fori_loop(0, NCH, chunk_body, 0)

    pltpu.sync_copy(acc_v, sums_hbm.at[pl.ds(lo, CPT)])
    pltpu.sync_copy(cnt_v, counts_hbm.at[pl.ds(lo, CPT)])


def _sc_segment(feat, lbl):
    mesh = plsc.VectorSubcoreMesh(core_axis_name="c", subcore_axis_name="s",
                                  num_cores=NC, num_subcores=NS)
    return pl.kernel(
        _sc_segment_body,
        out_type=(jax.ShapeDtypeStruct((NUM_CLASSES, EMB), jnp.float32),
                  jax.ShapeDtypeStruct((NUM_CLASSES, L), jnp.float32)),
        mesh=mesh,
        compiler_params=pltpu.CompilerParams(needs_layout_passes=False),
        scratch_types=[
            pltpu.VMEM((LCH,), jnp.int32),
            pltpu.VMEM((LCH,), jnp.int32),
            pltpu.VMEM((LCH,), jnp.int32),
            pltpu.VMEM((GB, EMB), jnp.float32),
            pltpu.VMEM((CPT, EMB), jnp.float32),
            pltpu.VMEM((CPT, L), jnp.float32),
            pltpu.SemaphoreType.DMA,
        ],
    )(feat, lbl)


def _proto_body(sums_ref, counts_ref, w1_ref, b1_ref, w2_ref, b2_ref, t_ref):
    sums = sums_ref[...]
    inv = 1.0 / jnp.maximum(counts_ref[:, 0:1], 1.0)
    protos = sums * inv
    h = jnp.dot(protos, w1_ref[...], preferred_element_type=jnp.float32)
    h = jnp.maximum(h + b1_ref[...], 0.0)
    t = jnp.dot(h, w2_ref[...], preferred_element_type=jnp.float32) + b2_ref[...]
    ss = jnp.sum(t * t, axis=1, keepdims=True)
    t_ref[...] = t * lax.rsqrt(jnp.maximum(ss, 1e-24))


def _proto_mlp(sums, counts, W1, b1, W2, b2):
    return pl.pallas_call(
        _proto_body,
        out_shape=jax.ShapeDtypeStruct((NUM_CLASSES, EMB), jnp.float32),
    )(sums, counts, W1, b1.reshape(1, EMB), W2, b2.reshape(1, EMB))


BQ = 2048


def _logits_body(q_ref, t_ref, out_ref):
    q = q_ref[...]
    qn = q * lax.rsqrt(jnp.maximum(jnp.sum(q * q, axis=1, keepdims=True), 1e-24))
    out_ref[...] = lax.dot_general(qn, t_ref[...], (((1,), (1,)), ((), ())),
                                   preferred_element_type=jnp.float32)


def _logits(q, t):
    return pl.pallas_call(
        _logits_body,
        grid=(N_QUERY // BQ,),
        in_specs=[
            pl.BlockSpec((BQ, EMB), lambda i: (i, 0)),
            pl.BlockSpec((NUM_CLASSES, EMB), lambda i: (0, 0)),
        ],
        out_specs=pl.BlockSpec((BQ, NUM_CLASSES), lambda i: (i, 0)),
        out_shape=jax.ShapeDtypeStruct((N_QUERY, NUM_CLASSES), jnp.float32),
    )(q, t)


def kernel(support_features, support_labels, query_features, W1, b1, W2, b2):
    lbl = support_labels.astype(jnp.int32)
    sums, counts = _sc_segment(support_features, lbl)
    t = _proto_mlp(sums, counts, W1, b1, W2, b2)
    logits = _logits(query_features, t)
    return logits, t


# speedup vs baseline: 2.0656x; 2.0656x over previous
import functools

import jax
import jax.numpy as jnp
from jax import lax
from jax.experimental import pallas as pl
from jax.experimental.pallas import tpu as pltpu
from jax.experimental.pallas import tpu_sc as plsc

NUM_CLASSES = 1024
EMB = 512
N_SUPPORT = 32768
N_QUERY = 16384

NC = 2
NS = 16
L = 16
NW = NC * NS
CPT = NUM_CLASSES // NW
CHUNKS = EMB // L
LCH = 4096
NCH = N_SUPPORT // LCH
GB = 64


def _sc_segment_body(feat_hbm, lbl_hbm, sums_hbm, counts_hbm,
                     lblchunk_v, ilist_v, batch0_v, batch1_v, lblb0_v, lblb1_v,
                     acc_v, cnt_v, semr0, semr1, seml0, seml1):
    c = lax.axis_index("c")
    s = lax.axis_index("s")
    tid = c * NS + s
    lo = tid * CPT

    zeros = jnp.zeros((L,), jnp.float32)

    def zero_acc(r, _):
        for k in range(CHUNKS):
            acc_v[r, pl.ds(k * L, L)] = zeros
        cnt_v[r, pl.ds(0, L)] = zeros
        return 0
    lax.fori_loop(0, CPT, zero_acc, 0)

    iota16 = lax.iota(jnp.int32, L)

    def chunk_body(ch, n):
        pltpu.sync_copy(lbl_hbm.at[pl.ds(ch * LCH, LCH)], lblchunk_v)

        def scan_body(i, n):
            lv = lblchunk_v[pl.ds(i * L, L)]
            m = (lv >= lo) & (lv < lo + CPT)
            plsc.store_compressed(ilist_v.at[pl.ds(n, L)],
                                  iota16 + (ch * LCH + i * L), mask=m)
            return n + plsc.all_reduce_population_count(m)[0]
        return lax.fori_loop(0, LCH // L, scan_body, n)

    n = lax.fori_loop(0, NCH, chunk_body, 0)

    for i in range(GB // L):
        ilist_v[pl.ds(n + i * L, L)] = jnp.zeros((L,), jnp.int32)

    nb = (n + (GB - 1)) // GB

    def issue(b, batch_ref, lblb_ref, semr, seml):
        sl = ilist_v.at[pl.ds(b * GB, GB)]
        pltpu.async_copy(feat_hbm.at[sl], batch_ref, semr)
        pltpu.async_copy(lbl_hbm.at[sl], lblb_ref, seml)

    def wait(b, batch_ref, lblb_ref, semr, seml):
        sl = ilist_v.at[pl.ds(b * GB, GB)]
        pltpu.make_async_copy(feat_hbm.at[sl], batch_ref, semr).wait()
        pltpu.make_async_copy(lbl_hbm.at[sl], lblb_ref, seml).wait()

    def process(b, batch_ref, lblb_ref):
        def group_body(g, _):
            lvec = lblb_ref[pl.ds(g * L, L)]
            wvec = jnp.where((iota16 + (b * GB + g * L)) < n,
                             jnp.float32(1.0), jnp.float32(0.0))
            for j in range(L):
                lab = lvec[j]
                cl = jnp.minimum(jnp.maximum(lab - lo, 0), CPT - 1)
                r = g * L + j
                vs = []
                parts = []
                for k in range(CHUNKS):
                    v = batch_ref[r, pl.ds(k * L, L)]
                    vs.append(v)
                    parts.append(v * v)
                while len(parts) > 1:
                    rest = [parts[-1]] if len(parts) % 2 else []
                    parts = [parts[i2] + parts[i2 + 1]
                             for i2 in range(0, len(parts) - 1, 2)] + rest
                tv = jnp.broadcast_to(jnp.sum(parts[0]), (L,))
                iv = plsc.bitcast(tv, jnp.int32)
                y = plsc.bitcast(jnp.int32(0x5F3759DF) - (iv >> 1),
                                 jnp.float32)
                for _ in range(3):
                    y = y * (jnp.float32(1.5) - jnp.float32(0.5) * tv * y * y)
                w = wvec[j]
                y = y * w
                for k in range(CHUNKS):
                    acc_v[cl, pl.ds(k * L, L)] = (
                        acc_v[cl, pl.ds(k * L, L)] + vs[k] * y)
                cnt_v[cl, pl.ds(0, L)] = (cnt_v[cl, pl.ds(0, L)]
                                          + jnp.broadcast_to(w, (L,)))
            return 0

        lax.fori_loop(0, GB // L, group_body, 0)

    @pl.when(nb > 0)
    def _():
        issue(0, batch0_v, lblb0_v, semr0, seml0)

    @pl.when(nb > 1)
    def _():
        issue(1, batch1_v, lblb1_v, semr1, seml1)

    def pair_body(p, _):
        b0 = 2 * p
        b1 = b0 + 1
        wait(b0, batch0_v, lblb0_v, semr0, seml0)
        process(b0, batch0_v, lblb0_v)

        @pl.when(b0 + 2 < nb)
        def _():
            issue(b0 + 2, batch0_v, lblb0_v, semr0, seml0)

        @pl.when(b1 < nb)
        def _():
            wait(b1, batch1_v, lblb1_v, semr1, seml1)
            process(b1, batch1_v, lblb1_v)

            @pl.when(b1 + 2 < nb)
            def _():
                issue(b1 + 2, batch1_v, lblb1_v, semr1, seml1)
        return 0

    lax.fori_loop(0, (nb + 1) // 2, pair_body, 0)

    pltpu.sync_copy(acc_v, sums_hbm.at[pl.ds(lo, CPT)])
    pltpu.sync_copy(cnt_v, counts_hbm.at[pl.ds(lo, CPT)])


def _sc_segment(feat, lbl):
    mesh = plsc.VectorSubcoreMesh(core_axis_name="c", subcore_axis_name="s",
                                  num_cores=NC, num_subcores=NS)
    return pl.kernel(
        _sc_segment_body,
        out_type=(jax.ShapeDtypeStruct((NUM_CLASSES, EMB), jnp.float32),
                  jax.ShapeDtypeStruct((NUM_CLASSES, L), jnp.float32)),
        mesh=mesh,
        compiler_params=pltpu.CompilerParams(needs_layout_passes=False),
        scratch_types=[
            pltpu.VMEM((LCH,), jnp.int32),
            pltpu.VMEM((N_SUPPORT + GB,), jnp.int32),
            pltpu.VMEM((GB, EMB), jnp.float32),
            pltpu.VMEM((GB, EMB), jnp.float32),
            pltpu.VMEM((GB,), jnp.int32),
            pltpu.VMEM((GB,), jnp.int32),
            pltpu.VMEM((CPT, EMB), jnp.float32),
            pltpu.VMEM((CPT, L), jnp.float32),
            pltpu.SemaphoreType.DMA,
            pltpu.SemaphoreType.DMA,
            pltpu.SemaphoreType.DMA,
            pltpu.SemaphoreType.DMA,
        ],
    )(feat, lbl)


def _proto_body(sums_ref, counts_ref, w1_ref, b1_ref, w2_ref, b2_ref, t_ref):
    sums = sums_ref[...]
    inv = 1.0 / jnp.maximum(counts_ref[:, 0:1], 1.0)
    protos = sums * inv
    h = jnp.dot(protos, w1_ref[...], preferred_element_type=jnp.float32)
    h = jnp.maximum(h + b1_ref[...], 0.0)
    t = jnp.dot(h, w2_ref[...], preferred_element_type=jnp.float32) + b2_ref[...]
    ss = jnp.sum(t * t, axis=1, keepdims=True)
    t_ref[...] = t * lax.rsqrt(jnp.maximum(ss, 1e-24))


def _proto_mlp(sums, counts, W1, b1, W2, b2):
    return pl.pallas_call(
        _proto_body,
        out_shape=jax.ShapeDtypeStruct((NUM_CLASSES, EMB), jnp.float32),
    )(sums, counts, W1, b1.reshape(1, EMB), W2, b2.reshape(1, EMB))


BQ = 2048


def _logits_body(q_ref, t_ref, out_ref):
    q = q_ref[...]
    qn = q * lax.rsqrt(jnp.maximum(jnp.sum(q * q, axis=1, keepdims=True), 1e-24))
    out_ref[...] = lax.dot_general(qn, t_ref[...], (((1,), (1,)), ((), ())),
                                   preferred_element_type=jnp.float32)


def _logits(q, t):
    return pl.pallas_call(
        _logits_body,
        grid=(N_QUERY // BQ,),
        in_specs=[
            pl.BlockSpec((BQ, EMB), lambda i: (i, 0)),
            pl.BlockSpec((NUM_CLASSES, EMB), lambda i: (0, 0)),
        ],
        out_specs=pl.BlockSpec((BQ, NUM_CLASSES), lambda i: (i, 0)),
        out_shape=jax.ShapeDtypeStruct((N_QUERY, NUM_CLASSES), jnp.float32),
    )(q, t)


def kernel(support_features, support_labels, query_features, W1, b1, W2, b2):
    lbl = support_labels.astype(jnp.int32)
    sums, counts = _sc_segment(support_features, lbl)
    t = _proto_mlp(sums, counts, W1, b1, W2, b2)
    logits = _logits(query_features, t)
    return logits, t


# speedup vs baseline: 4.4770x; 2.1674x over previous
import functools

import jax
import jax.numpy as jnp
from jax import lax
from jax.experimental import pallas as pl
from jax.experimental.pallas import tpu as pltpu
from jax.experimental.pallas import tpu_sc as plsc

NUM_CLASSES = 1024
EMB = 512
N_SUPPORT = 32768
N_QUERY = 16384

NC = 2
NS = 16
L = 16
NW = NC * NS
ROWS_PER_TILE = N_SUPPORT // NW
CHUNKS = EMB // L
GB = 32
NBATCH = ROWS_PER_TILE // GB
NBUF = 4
CLS_STRIPE = NUM_CLASSES // NS
CW = 128


def _sc_segment_body(feat_hbm, lbl_hbm, sums_hbm, counts_hbm,
                     in0, in1, in2, in3, lb0, lb1, lb2, lb3, ones_v,
                     sums_sh, counts_sh,
                     sg0, sg1, sg2, sg3, ss0, ss1, ss2, ss3):
    c = lax.axis_index("c")
    s = lax.axis_index("s")
    tid = c * NS + s
    row_base = tid * ROWS_PER_TILE
    acc_base = c * NUM_CLASSES

    ins = [in0, in1, in2, in3]
    lbs = [lb0, lb1, lb2, lb3]
    sgs = [sg0, sg1, sg2, sg3]
    sss = [ss0, ss1, ss2, ss3]

    zeros = jnp.zeros((L,), jnp.float32)

    def zero_in0(r, _):
        for k in range(CHUNKS):
            in0[r, pl.ds(k * L, L)] = zeros
        for k in range(CW // L):
            ones_v[r, pl.ds(k * L, L)] = zeros
        return 0
    lax.fori_loop(0, GB, zero_in0, 0)
    stripe = s * CLS_STRIPE
    pltpu.sync_copy(in0, sums_sh.at[pl.ds(stripe, GB)])
    pltpu.sync_copy(in0, sums_sh.at[pl.ds(stripe + GB, GB)])
    pltpu.sync_copy(ones_v, counts_sh.at[pl.ds(stripe, GB)])
    pltpu.sync_copy(ones_v, counts_sh.at[pl.ds(stripe + GB, GB)])

    def fill_ones(r, _):
        for k in range(CW // L):
            ones_v[r, pl.ds(k * L, L)] = jnp.full((L,), 1.0, jnp.float32)
        return 0
    lax.fori_loop(0, GB, fill_ones, 0)

    plsc.subcore_barrier()

    def issue_gather(b, B):
        sl = pl.ds(row_base + b * GB, GB)
        pltpu.async_copy(feat_hbm.at[sl], ins[B], sgs[B])
        pltpu.async_copy(lbl_hbm.at[sl], lbs[B], sgs[B])

    def wait_gather(b, B):
        sl = pl.ds(row_base + b * GB, GB)
        pltpu.make_async_copy(feat_hbm.at[sl], ins[B], sgs[B]).wait()
        pltpu.make_async_copy(lbl_hbm.at[sl], lbs[B], sgs[B]).wait()

    def issue_scatter(B):
        pltpu.async_copy(ins[B], sums_sh.at[lbs[B]], sss[B], add=True)
        pltpu.async_copy(ones_v, counts_sh.at[lbs[B]], sss[B], add=True)

    def wait_scatter(B):
        pltpu.make_async_copy(ins[B], sums_sh.at[lbs[B]], sss[B]).wait()
        pltpu.make_async_copy(ones_v, counts_sh.at[lbs[B]], sss[B]).wait()

    def normalize(B):

        def rows_body(g, _):
            for j in range(4):
                r = g * 4 + j
                vs = []
                parts = []
                for k in range(CHUNKS):
                    v = ins[B][r, pl.ds(k * L, L)]
                    vs.append(v)
                    parts.append(v * v)
                while len(parts) > 1:
                    rest = [parts[-1]] if len(parts) % 2 else []
                    parts = [parts[i2] + parts[i2 + 1]
                             for i2 in range(0, len(parts) - 1, 2)] + rest
                tv = jnp.broadcast_to(jnp.sum(parts[0]), (L,))
                iv = plsc.bitcast(tv, jnp.int32)
                y = plsc.bitcast(jnp.int32(0x5F3759DF) - (iv >> 1),
                                 jnp.float32)
                for _ in range(3):
                    y = y * (jnp.float32(1.5) - jnp.float32(0.5) * tv * y * y)
                for k in range(CHUNKS):
                    ins[B][r, pl.ds(k * L, L)] = vs[k] * y
            return 0

        lax.fori_loop(0, GB // 4, rows_body, 0)

    issue_gather(0, 0)
    issue_gather(1, 1)
    issue_gather(2, 2)

    def quad_body(q, _):
        for sec in range(NBUF):
            b = q * NBUF + sec
            wait_gather(b, sec)
            normalize(sec)
            issue_scatter(sec)

            @pl.when(b >= 1)
            def _(sec=sec):
                wait_scatter((sec - 1) % NBUF)

            @pl.when(b + 3 < NBATCH)
            def _(b=b, sec=sec):
                issue_gather(b + 3, (sec + 3) % NBUF)
        return 0

    lax.fori_loop(0, NBATCH // NBUF, quad_body, 0)
    wait_scatter((NBATCH - 1) % NBUF)

    plsc.subcore_barrier()
    pltpu.sync_copy(sums_sh.at[pl.ds(stripe, CLS_STRIPE)],
                    sums_hbm.at[pl.ds(acc_base + stripe, CLS_STRIPE)])
    pltpu.sync_copy(counts_sh.at[pl.ds(stripe, CLS_STRIPE)],
                    counts_hbm.at[pl.ds(acc_base + stripe, CLS_STRIPE)])


def _sc_segment(feat, lbl):
    mesh = plsc.VectorSubcoreMesh(core_axis_name="c", subcore_axis_name="s",
                                  num_cores=NC, num_subcores=NS)
    return pl.kernel(
        _sc_segment_body,
        out_type=(jax.ShapeDtypeStruct((NC * NUM_CLASSES, EMB), jnp.float32),
                  jax.ShapeDtypeStruct((NC * NUM_CLASSES, CW), jnp.float32)),
        mesh=mesh,
        compiler_params=pltpu.CompilerParams(needs_layout_passes=False,
                                             use_tc_tiling_on_sc=False),
        scratch_types=(
            [pltpu.VMEM((GB, EMB), jnp.float32) for _ in range(NBUF)]
            + [pltpu.VMEM((GB,), jnp.int32) for _ in range(NBUF)]
            + [pltpu.VMEM((GB, CW), jnp.float32)]
            + [pltpu.VMEM_SHARED((NUM_CLASSES, EMB), jnp.float32),
               pltpu.VMEM_SHARED((NUM_CLASSES, CW), jnp.float32)]
            + [pltpu.SemaphoreType.DMA for _ in range(2 * NBUF)]
        ),
    )(feat, lbl)


def _proto_body(sums_ref, counts_ref, w1_ref, b1_ref, w2_ref, b2_ref, t_ref):
    sums = sums_ref[:NUM_CLASSES, :] + sums_ref[NUM_CLASSES:, :]
    cnt = counts_ref[:NUM_CLASSES, :] + counts_ref[NUM_CLASSES:, :]
    inv = 1.0 / jnp.maximum(cnt[:, 0:1], 1.0)
    protos = sums * inv
    h = jnp.dot(protos, w1_ref[...], preferred_element_type=jnp.float32)
    h = jnp.maximum(h + b1_ref[...], 0.0)
    t = jnp.dot(h, w2_ref[...], preferred_element_type=jnp.float32) + b2_ref[...]
    ss = jnp.sum(t * t, axis=1, keepdims=True)
    t_ref[...] = t * lax.rsqrt(jnp.maximum(ss, 1e-24))


def _proto_mlp(sums, counts, W1, b1, W2, b2):
    return pl.pallas_call(
        _proto_body,
        out_shape=jax.ShapeDtypeStruct((NUM_CLASSES, EMB), jnp.float32),
    )(sums, counts, W1, b1.reshape(1, EMB), W2, b2.reshape(1, EMB))


BQ = 2048


def _logits_body(q_ref, t_ref, out_ref):
    q = q_ref[...]
    qn = q * lax.rsqrt(jnp.maximum(jnp.sum(q * q, axis=1, keepdims=True), 1e-24))
    out_ref[...] = lax.dot_general(qn, t_ref[...], (((1,), (1,)), ((), ())),
                                   preferred_element_type=jnp.float32)


def _logits(q, t):
    return pl.pallas_call(
        _logits_body,
        grid=(N_QUERY // BQ,),
        in_specs=[
            pl.BlockSpec((BQ, EMB), lambda i: (i, 0)),
            pl.BlockSpec((NUM_CLASSES, EMB), lambda i: (0, 0)),
        ],
        out_specs=pl.BlockSpec((BQ, NUM_CLASSES), lambda i: (i, 0)),
        out_shape=jax.ShapeDtypeStruct((N_QUERY, NUM_CLASSES), jnp.float32),
    )(q, t)


def kernel(support_features, support_labels, query_features, W1, b1, W2, b2):
    lbl = support_labels.astype(jnp.int32)
    sums, counts = _sc_segment(support_features, lbl)
    t = _proto_mlp(sums, counts, W1, b1, W2, b2)
    logits = _logits(query_features, t)
    return logits, t


# speedup vs baseline: 5.4685x; 1.2215x over previous
import functools

import jax
import jax.numpy as jnp
from jax import lax
from jax.experimental import pallas as pl
from jax.experimental.pallas import tpu as pltpu
from jax.experimental.pallas import tpu_sc as plsc

NUM_CLASSES = 1024
EMB = 512
N_SUPPORT = 32768
N_QUERY = 16384

NC = 2
NS = 16
L = 16
NW = NC * NS
ROWS_PER_TILE = N_SUPPORT // NW
CHUNKS = EMB // L
GB = 32
NBATCH = ROWS_PER_TILE // GB
NBUF = 4
CLS_STRIPE = NUM_CLASSES // NS
CW = 128


def _sc_segment_body(feat_hbm, lbl_hbm, sums_hbm, counts_hbm,
                     in0, in1, out0, out1, lbi0, lbi1, lbo0, lbo1, ones_v,
                     sums_sh, counts_sh, sg0, sg1, ss0, ss1):
    c = lax.axis_index("c")
    s = lax.axis_index("s")
    tid = c * NS + s
    row_base = tid * ROWS_PER_TILE
    acc_base = c * NUM_CLASSES

    ins = [in0, in1]
    outs = [out0, out1]
    lbis = [lbi0, lbi1]
    lbos = [lbo0, lbo1]
    sgs = [sg0, sg1]
    sss = [ss0, ss1]

    zeros = jnp.zeros((L,), jnp.float32)

    def zero_out0(r, _):
        for k in range(CHUNKS):
            out0[r, pl.ds(k * L, L)] = zeros
        for k in range(CW // L):
            ones_v[r, pl.ds(k * L, L)] = zeros
        return 0
    lax.fori_loop(0, GB, zero_out0, 0)
    stripe = s * CLS_STRIPE
    pltpu.sync_copy(out0, sums_sh.at[pl.ds(stripe, GB)])
    pltpu.sync_copy(out0, sums_sh.at[pl.ds(stripe + GB, GB)])
    pltpu.sync_copy(ones_v, counts_sh.at[pl.ds(stripe, GB)])
    pltpu.sync_copy(ones_v, counts_sh.at[pl.ds(stripe + GB, GB)])

    def fill_ones(r, _):
        for k in range(CW // L):
            ones_v[r, pl.ds(k * L, L)] = jnp.full((L,), 1.0, jnp.float32)
        return 0
    lax.fori_loop(0, GB, fill_ones, 0)

    plsc.subcore_barrier()

    def issue_gather(b, B):
        blk = (row_base + b * GB) // 8
        pltpu.async_copy(feat_hbm.at[pl.ds(blk, GB // 8)], ins[B], sgs[B])
        pltpu.async_copy(lbl_hbm.at[pl.ds(row_base + b * GB, GB)],
                         lbis[B], sgs[B])

    def wait_gather(b, B):
        blk = (row_base + b * GB) // 8
        pltpu.make_async_copy(feat_hbm.at[pl.ds(blk, GB // 8)],
                              ins[B], sgs[B]).wait()
        pltpu.make_async_copy(lbl_hbm.at[pl.ds(row_base + b * GB, GB)],
                              lbis[B], sgs[B]).wait()

    def issue_scatter(B):
        pltpu.async_copy(outs[B], sums_sh.at[lbos[B]], sss[B], add=True)
        pltpu.async_copy(ones_v, counts_sh.at[lbos[B]], sss[B], add=True)

    def wait_scatter(B):
        pltpu.make_async_copy(outs[B], sums_sh.at[lbos[B]], sss[B]).wait()
        pltpu.make_async_copy(ones_v, counts_sh.at[lbos[B]], sss[B]).wait()

    def normalize(B):
        for g in range(GB // L):
            lbos[B][pl.ds(g * L, L)] = lbis[B][pl.ds(g * L, L)]

        def rows_body(g, _):
            for jj in range(4):
                r = g * 4 + jj
                rb = r // 8
                rs = r % 8
                vs = []
                parts = []
                for k in range(CHUNKS):
                    v = ins[B][rb, k // 8, rs, pl.ds((k % 8) * L, L)]
                    vs.append(v)
                    parts.append(v * v)
                while len(parts) > 1:
                    rest = [parts[-1]] if len(parts) % 2 else []
                    parts = [parts[i2] + parts[i2 + 1]
                             for i2 in range(0, len(parts) - 1, 2)] + rest
                tv = jnp.broadcast_to(jnp.sum(parts[0]), (L,))
                iv = plsc.bitcast(tv, jnp.int32)
                y = plsc.bitcast(jnp.int32(0x5F3759DF) - (iv >> 1),
                                 jnp.float32)
                for _ in range(3):
                    y = y * (jnp.float32(1.5) - jnp.float32(0.5) * tv * y * y)
                for k in range(CHUNKS):
                    outs[B][r, pl.ds(k * L, L)] = vs[k] * y
            return 0

        lax.fori_loop(0, GB // 4, rows_body, 0)

    issue_gather(0, 0)
    issue_gather(1, 1)

    def pair_body(p, _):
        for B in range(2):
            b = 2 * p + B
            wait_gather(b, B)

            @pl.when(b >= 2)
            def _(B=B):
                wait_scatter(B)
            normalize(B)
            issue_scatter(B)

            @pl.when(b + 2 < NBATCH)
            def _(b=b, B=B):
                issue_gather(b + 2, B)
        return 0

    lax.fori_loop(0, NBATCH // 2, pair_body, 0)
    wait_scatter(0)
    wait_scatter(1)

    plsc.subcore_barrier()
    pltpu.sync_copy(sums_sh.at[pl.ds(stripe, CLS_STRIPE)],
                    sums_hbm.at[pl.ds(acc_base + stripe, CLS_STRIPE)])
    pltpu.sync_copy(counts_sh.at[pl.ds(stripe, CLS_STRIPE)],
                    counts_hbm.at[pl.ds(acc_base + stripe, CLS_STRIPE)])


def _sc_segment(feat_t, lbl):
    mesh = plsc.VectorSubcoreMesh(core_axis_name="c", subcore_axis_name="s",
                                  num_cores=NC, num_subcores=NS)
    return pl.kernel(
        _sc_segment_body,
        out_type=(jax.ShapeDtypeStruct((NC * NUM_CLASSES, EMB), jnp.float32),
                  jax.ShapeDtypeStruct((NC * NUM_CLASSES, CW), jnp.float32)),
        mesh=mesh,
        compiler_params=pltpu.CompilerParams(needs_layout_passes=False,
                                             use_tc_tiling_on_sc=False),
        scratch_types=(
            [pltpu.VMEM((GB // 8, EMB // 128, 8, 128), jnp.float32)
             for _ in range(2)]
            + [pltpu.VMEM((GB, EMB), jnp.float32) for _ in range(2)]
            + [pltpu.VMEM((GB,), jnp.int32) for _ in range(4)]
            + [pltpu.VMEM((GB, CW), jnp.float32)]
            + [pltpu.VMEM_SHARED((NUM_CLASSES, EMB), jnp.float32),
               pltpu.VMEM_SHARED((NUM_CLASSES, CW), jnp.float32)]
            + [pltpu.SemaphoreType.DMA for _ in range(4)]
        ),
    )(feat_t, lbl)


def _proto_body(sums_ref, counts_ref, w1_ref, b1_ref, w2_ref, b2_ref, t_ref):
    sums = sums_ref[:NUM_CLASSES, :] + sums_ref[NUM_CLASSES:, :]
    cnt = counts_ref[:NUM_CLASSES, :] + counts_ref[NUM_CLASSES:, :]
    inv = 1.0 / jnp.maximum(cnt[:, 0:1], 1.0)
    protos = sums * inv
    h = jnp.dot(protos, w1_ref[...], preferred_element_type=jnp.float32)
    h = jnp.maximum(h + b1_ref[...], 0.0)
    t = jnp.dot(h, w2_ref[...], preferred_element_type=jnp.float32) + b2_ref[...]
    ss = jnp.sum(t * t, axis=1, keepdims=True)
    t_ref[...] = t * lax.rsqrt(jnp.maximum(ss, 1e-24))


def _proto_mlp(sums, counts, W1, b1, W2, b2):
    return pl.pallas_call(
        _proto_body,
        out_shape=jax.ShapeDtypeStruct((NUM_CLASSES, EMB), jnp.float32),
    )(sums, counts, W1, b1.reshape(1, EMB), W2, b2.reshape(1, EMB))


BQ = 2048


def _logits_body(q_ref, t_ref, out_ref):
    q = q_ref[...]
    qn = q * lax.rsqrt(jnp.maximum(jnp.sum(q * q, axis=1, keepdims=True), 1e-24))
    out_ref[...] = lax.dot_general(qn, t_ref[...], (((1,), (1,)), ((), ())),
                                   preferred_element_type=jnp.float32)


def _logits(q, t):
    return pl.pallas_call(
        _logits_body,
        grid=(N_QUERY // BQ,),
        in_specs=[
            pl.BlockSpec((BQ, EMB), lambda i: (i, 0)),
            pl.BlockSpec((NUM_CLASSES, EMB), lambda i: (0, 0)),
        ],
        out_specs=pl.BlockSpec((BQ, NUM_CLASSES), lambda i: (i, 0)),
        out_shape=jax.ShapeDtypeStruct((N_QUERY, NUM_CLASSES), jnp.float32),
    )(q, t)


def kernel(support_features, support_labels, query_features, W1, b1, W2, b2):
    lbl = support_labels.astype(jnp.int32)
    feat_t = support_features.reshape(N_SUPPORT // 8, 8, EMB // 128,
                                      128).swapaxes(1, 2)
    sums, counts = _sc_segment(feat_t, lbl)
    t = _proto_mlp(sums, counts, W1, b1, W2, b2)
    logits = _logits(query_features, t)
    return logits, t


# speedup vs baseline: 6.3804x; 1.1668x over previous
import functools

import jax
import jax.numpy as jnp
from jax import lax
from jax.experimental import pallas as pl
from jax.experimental.pallas import tpu as pltpu
from jax.experimental.pallas import tpu_sc as plsc

NUM_CLASSES = 1024
EMB = 512
N_SUPPORT = 32768
N_QUERY = 16384

NC = 2
NS = 16
L = 16
NW = NC * NS
ROWS_PER_TILE = N_SUPPORT // NW
CHUNKS = EMB // L
GB = 32
NBATCH = ROWS_PER_TILE // GB
NBUF = 4
CLS_STRIPE = NUM_CLASSES // NS
CW = 16


def _sc_segment_body(feat_hbm, lbl_hbm, sums_hbm, counts_hbm,
                     in0, in1, out0, out1,
                     lbi0, lbi1, lbo0, lbo1, ones_v,
                     sums_sh, counts_sh, sg0, sg1, ss0, ss1):
    c = lax.axis_index("c")
    s = lax.axis_index("s")
    tid = c * NS + s
    row_base = tid * ROWS_PER_TILE
    acc_base = c * NUM_CLASSES

    ins = [in0, in1]
    outs = [out0, out1]
    lbis = [lbi0, lbi1]
    lbos = [lbo0, lbo1]
    sgs = [sg0, sg1]
    sss = [ss0, ss1]

    zeros = jnp.zeros((L,), jnp.float32)

    def zero_out0(r, _):
        for k in range(CHUNKS):
            out0[r, pl.ds(k * L, L)] = zeros
        for k in range(CW // L):
            ones_v[r, pl.ds(k * L, L)] = zeros
        return 0
    lax.fori_loop(0, GB, zero_out0, 0)
    stripe = s * CLS_STRIPE
    pltpu.sync_copy(out0, sums_sh.at[pl.ds(stripe, GB)])
    pltpu.sync_copy(out0, sums_sh.at[pl.ds(stripe + GB, GB)])
    pltpu.sync_copy(ones_v, counts_sh.at[pl.ds(stripe, GB)])
    pltpu.sync_copy(ones_v, counts_sh.at[pl.ds(stripe + GB, GB)])

    def fill_ones(r, _):
        for k in range(CW // L):
            ones_v[r, pl.ds(k * L, L)] = jnp.full((L,), 1.0, jnp.float32)
        return 0
    lax.fori_loop(0, GB, fill_ones, 0)

    plsc.subcore_barrier()

    def issue_gather(b, B):
        blk = (row_base + b * GB) // 8
        pltpu.async_copy(feat_hbm.at[pl.ds(blk, GB // 8)], ins[B], sgs[B])
        pltpu.async_copy(lbl_hbm.at[pl.ds(row_base + b * GB, GB)],
                         lbis[B], sgs[B])

    def wait_gather(b, B):
        blk = (row_base + b * GB) // 8
        pltpu.make_async_copy(feat_hbm.at[pl.ds(blk, GB // 8)],
                              ins[B], sgs[B]).wait()
        pltpu.make_async_copy(lbl_hbm.at[pl.ds(row_base + b * GB, GB)],
                              lbis[B], sgs[B]).wait()

    def issue_scatter(B):
        pltpu.async_copy(outs[B], sums_sh.at[lbos[B]], sss[B], add=True)
        pltpu.async_copy(ones_v, counts_sh.at[lbos[B]], sss[B], add=True)

    def wait_scatter(B):
        pltpu.make_async_copy(outs[B], sums_sh.at[lbos[B]], sss[B]).wait()
        pltpu.make_async_copy(ones_v, counts_sh.at[lbos[B]], sss[B]).wait()

    def normalize(BI, BO):
        for g in range(GB // L):
            lbos[BO][pl.ds(g * L, L)] = lbis[BI][pl.ds(g * L, L)]

        def block_body(rb, _):
            for rs in range(8):
                r = rb * 8 + rs
                vs = []
                parts = []
                for k in range(CHUNKS):
                    v = ins[BI][rb, k // 8, rs, pl.ds((k % 8) * L, L)]
                    vs.append(v)
                    parts.append(v * v)
                while len(parts) > 1:
                    rest = [parts[-1]] if len(parts) % 2 else []
                    parts = [parts[i2] + parts[i2 + 1]
                             for i2 in range(0, len(parts) - 1, 2)] + rest
                tv = jnp.broadcast_to(jnp.sum(parts[0]), (L,))
                iv = plsc.bitcast(tv, jnp.int32)
                y = plsc.bitcast(jnp.int32(0x5F3759DF) - (iv >> 1),
                                 jnp.float32)
                for _ in range(3):
                    y = y * (jnp.float32(1.5) - jnp.float32(0.5) * tv * y * y)
                for k in range(CHUNKS):
                    outs[BO][r, pl.ds(k * L, L)] = vs[k] * y
            return 0

        lax.fori_loop(0, GB // 8, block_body, 0)

    issue_gather(0, 0)
    issue_gather(1, 1)

    def pair_body(p, _):
        for sec in range(2):
            b = 2 * p + sec
            wait_gather(b, sec)

            @pl.when(b >= 2)
            def _(sec=sec):
                wait_scatter(sec)
            normalize(sec, sec)
            issue_scatter(sec)

            @pl.when(b + 2 < NBATCH)
            def _(b=b, sec=sec):
                issue_gather(b + 2, sec)
        return 0

    lax.fori_loop(0, NBATCH // 2, pair_body, 0)
    wait_scatter(0)
    wait_scatter(1)

    plsc.subcore_barrier()
    pltpu.sync_copy(sums_sh.at[pl.ds(stripe, CLS_STRIPE)],
                    sums_hbm.at[pl.ds(acc_base + stripe, CLS_STRIPE)])
    pltpu.sync_copy(counts_sh.at[pl.ds(stripe, CLS_STRIPE)],
                    counts_hbm.at[pl.ds(acc_base + stripe, CLS_STRIPE)])


def _sc_segment(feat_t, lbl):
    mesh = plsc.VectorSubcoreMesh(core_axis_name="c", subcore_axis_name="s",
                                  num_cores=NC, num_subcores=NS)
    return pl.kernel(
        _sc_segment_body,
        out_type=(jax.ShapeDtypeStruct((NC * NUM_CLASSES, EMB), jnp.float32),
                  jax.ShapeDtypeStruct((NC * NUM_CLASSES, CW), jnp.float32)),
        mesh=mesh,
        compiler_params=pltpu.CompilerParams(needs_layout_passes=False,
                                             use_tc_tiling_on_sc=False),
        scratch_types=(
            [pltpu.VMEM((GB // 8, EMB // 128, 8, 128), jnp.float32)
             for _ in range(2)]
            + [pltpu.VMEM((GB, EMB), jnp.float32) for _ in range(2)]
            + [pltpu.VMEM((GB,), jnp.int32) for _ in range(4)]
            + [pltpu.VMEM((GB, CW), jnp.float32)]
            + [pltpu.VMEM_SHARED((NUM_CLASSES, EMB), jnp.float32),
               pltpu.VMEM_SHARED((NUM_CLASSES, CW), jnp.float32)]
            + [pltpu.SemaphoreType.DMA for _ in range(4)]
        ),
    )(feat_t, lbl)


def _proto_body(sums_ref, counts_ref, w1_ref, b1_ref, w2_ref, b2_ref, t_ref):
    sums = sums_ref[:NUM_CLASSES, :] + sums_ref[NUM_CLASSES:, :]
    cnt = counts_ref[:NUM_CLASSES, :] + counts_ref[NUM_CLASSES:, :]
    inv = 1.0 / jnp.maximum(cnt[:, 0:1], 1.0)
    protos = sums * inv
    h = jnp.dot(protos, w1_ref[...], preferred_element_type=jnp.float32)
    h = jnp.maximum(h + b1_ref[...], 0.0)
    t = jnp.dot(h, w2_ref[...], preferred_element_type=jnp.float32) + b2_ref[...]
    ss = jnp.sum(t * t, axis=1, keepdims=True)
    t_ref[...] = t * lax.rsqrt(jnp.maximum(ss, 1e-24))


def _proto_mlp(sums, counts, W1, b1, W2, b2):
    return pl.pallas_call(
        _proto_body,
        out_shape=jax.ShapeDtypeStruct((NUM_CLASSES, EMB), jnp.float32),
    )(sums, counts, W1, b1.reshape(1, EMB), W2, b2.reshape(1, EMB))


BQ = 2048


def _logits_body(q_ref, t_ref, out_ref):
    q = q_ref[...]
    qn = q * lax.rsqrt(jnp.maximum(jnp.sum(q * q, axis=1, keepdims=True), 1e-24))
    out_ref[...] = lax.dot_general(qn, t_ref[...], (((1,), (1,)), ((), ())),
                                   preferred_element_type=jnp.float32)


def _logits(q, t):
    return pl.pallas_call(
        _logits_body,
        grid=(N_QUERY // BQ,),
        in_specs=[
            pl.BlockSpec((BQ, EMB), lambda i: (i, 0)),
            pl.BlockSpec((NUM_CLASSES, EMB), lambda i: (0, 0)),
        ],
        out_specs=pl.BlockSpec((BQ, NUM_CLASSES), lambda i: (i, 0)),
        out_shape=jax.ShapeDtypeStruct((N_QUERY, NUM_CLASSES), jnp.float32),
    )(q, t)


def kernel(support_features, support_labels, query_features, W1, b1, W2, b2):
    lbl = support_labels.astype(jnp.int32)
    feat_t = support_features.reshape(N_SUPPORT // 8, 8, EMB // 128,
                                      128).swapaxes(1, 2)
    sums, counts = _sc_segment(feat_t, lbl)
    t = _proto_mlp(sums, counts, W1, b1, W2, b2)
    logits = _logits(query_features, t)
    return logits, t
